# Initial kernel scaffold; baseline (speedup 1.0000x reference)
import jax
import jax.numpy as jnp
from jax.experimental import pallas as pl


def kernel(x, edge_index, v, a, b, W1_0, b1_0, W2_0, b2_0, eps_0, W1_1, b1_1, W2_1, b2_1, eps_1, P1, pb1, P2, pb2):
    raise NotImplementedError("write your pallas kernel here")


# speedup vs baseline: 4.1192x; 4.1192x over previous
import functools

import jax
import jax.numpy as jnp
from jax import lax
from jax.experimental import pallas as pl
from jax.experimental.pallas import tpu as pltpu
from jax.experimental.pallas import tpu_sc as plsc

_N = 10000
_D = 128
_E = 320000
_B = 1024
_TAU = 0.2

_NC = 2
_NS = 16
_NW = _NC * _NS
_K = 128
_CH = 79
_EPW = _CH * _K
_NPAD = 10240
_ZR = 128
_RPS = _NPAD // _NS
_GPW = 3 * _B // _NW

@functools.cache
def _make_segment_sum():
    mesh = plsc.VectorSubcoreMesh(core_axis_name="c", subcore_axis_name="s",
                                  num_cores=_NC, num_subcores=_NS)
    return functools.partial(
        pl.kernel,
        out_type=jax.ShapeDtypeStruct((_NC, _NPAD, _D), jnp.float32),
        mesh=mesh,
        scratch_types=[
            pltpu.VMEM((_CH, _K), jnp.int32),
            pltpu.VMEM((_CH, _K), jnp.int32),
            pltpu.VMEM((_K, _D), jnp.float32),
            pltpu.VMEM_SHARED((_NPAD, _D), jnp.float32),
            pltpu.SemaphoreType.DMA,
        ],
    )(_segment_sum_body)


def _segment_sum_body(table, srcw, dstw, zeros_hbm, out, src_v, dst_v, rows_v,
                      acc, sem):
    c = lax.axis_index("c")
    s = lax.axis_index("s")
    wid = s * _NC + c
    for t in range(_RPS // _ZR):
        pltpu.sync_copy(zeros_hbm, acc.at[pl.ds(s * _RPS + t * _ZR, _ZR)])
    pltpu.sync_copy(srcw.at[wid], src_v)
    pltpu.sync_copy(dstw.at[wid], dst_v)
    plsc.subcore_barrier()

    def body(j, carry):
        pltpu.async_copy(table.at[src_v.at[j]], rows_v, sem).wait()
        pltpu.sync_copy(rows_v, acc.at[dst_v.at[j]], add=True)
        return carry

    lax.fori_loop(0, _CH, body, 0)
    plsc.subcore_barrier()
    pltpu.sync_copy(acc.at[pl.ds(s * _RPS, _RPS)],
                    out.at[c, pl.ds(s * _RPS, _RPS)])


@functools.cache
def _make_gather_rows():
    mesh = plsc.VectorSubcoreMesh(core_axis_name="c", subcore_axis_name="s",
                                  num_cores=_NC, num_subcores=_NS)
    return functools.partial(
        pl.kernel,
        out_type=jax.ShapeDtypeStruct((3 * _B, _D), jnp.float32),
        mesh=mesh,
        scratch_types=[
            pltpu.VMEM((_GPW,), jnp.int32),
            pltpu.VMEM((_GPW, _D), jnp.float32),
            pltpu.SemaphoreType.DMA,
        ],
    )(_gather_rows_body)


def _gather_rows_body(table, idx, out, idx_v, rows_v, sem):
    c = lax.axis_index("c")
    s = lax.axis_index("s")
    base = (s * _NC + c) * _GPW
    pltpu.sync_copy(idx.at[pl.ds(base, _GPW)], idx_v)
    pltpu.async_copy(table.at[idx_v], rows_v, sem).wait()
    pltpu.sync_copy(rows_v, out.at[pl.ds(base, _GPW)])


def _mlp_block(eps_ref, x_ref, pa_ref, pb_ref, w1_ref, b1_ref, w2_ref,
               b2_ref, o_ref):
    z = (1.0 + eps_ref[0]) * x_ref[...] + pa_ref[0] + pb_ref[0]
    z = jnp.maximum(
        jnp.dot(z, w1_ref[...], preferred_element_type=jnp.float32)
        + b1_ref[...], 0.0)
    z = jnp.dot(z, w2_ref[...], preferred_element_type=jnp.float32) \
        + b2_ref[...]
    o_ref[...] = jnp.maximum(z, 0.0)


def _gin_mlp(x, parts, W1, b1, W2, b2, eps):
    R = 1000
    return pl.pallas_call(
        _mlp_block,
        grid=(_N // R,),
        in_specs=[
            pl.BlockSpec(memory_space=pltpu.SMEM),
            pl.BlockSpec((R, _D), lambda i: (i, 0)),
            pl.BlockSpec((1, R, _D), lambda i: (0, i, 0)),
            pl.BlockSpec((1, R, _D), lambda i: (1, i, 0)),
            pl.BlockSpec((_D, _D), lambda i: (0, 0)),
            pl.BlockSpec((1, _D), lambda i: (0, 0)),
            pl.BlockSpec((_D, _D), lambda i: (0, 0)),
            pl.BlockSpec((1, _D), lambda i: (0, 0)),
        ],
        out_specs=pl.BlockSpec((R, _D), lambda i: (i, 0)),
        out_shape=jax.ShapeDtypeStruct((_N, _D), jnp.float32),
    )(eps.reshape(1), x, parts, parts, W1, b1.reshape(1, _D), W2,
      b2.reshape(1, _D))


def _head_block(g_ref, p1_ref, pb1_ref, p2_ref, pb2_ref, o_ref):
    z = jnp.maximum(
        jnp.dot(g_ref[...], p1_ref[...], preferred_element_type=jnp.float32)
        + pb1_ref[...], 0.0)
    z = jnp.dot(z, p2_ref[...], preferred_element_type=jnp.float32) \
        + pb2_ref[...]
    sv = z[0:_B]
    sa = z[_B:2 * _B]
    sb = z[2 * _B:3 * _B]

    def cos(u, w):
        un = jnp.sqrt(jnp.sum(u * u, axis=1, keepdims=True))
        wn = jnp.sqrt(jnp.sum(w * w, axis=1, keepdims=True))
        return jnp.sum(u * w, axis=1, keepdims=True) / jnp.maximum(
            un * wn, 1e-8)

    pos = cos(sv, sa)
    neg = cos(sv, sb)
    num = jnp.exp(pos / _TAU)
    den = num + jnp.exp(neg / _TAU)
    o_ref[0, 0] = -jnp.sum(jnp.log(num / den)) / _B


def _head(g, P1, pb1, P2, pb2):
    return pl.pallas_call(
        _head_block,
        out_shape=jax.ShapeDtypeStruct((1, 1), jnp.float32),
        out_specs=pl.BlockSpec(memory_space=pltpu.SMEM),
    )(g, P1, pb1.reshape(1, _D), P2, pb2.reshape(1, _D))


def kernel(x, edge_index, v, a, b, W1_0, b1_0, W2_0, b2_0, eps_0, W1_1,
           b1_1, W2_1, b2_1, eps_1, P1, pb1, P2, pb2):
    src = edge_index[0]
    dst = edge_index[1]
    pad = _NW * _EPW - _E
    srcw = jnp.concatenate(
        [src, jnp.zeros((pad,), jnp.int32)]).reshape(_NW, _CH, _K)
    dstw = jnp.concatenate(
        [dst, jnp.full((pad,), _N, jnp.int32)]).reshape(_NW, _CH, _K)
    zrows = jnp.zeros((_ZR, _D), jnp.float32)

    segsum = _make_segment_sum()
    p0 = segsum(x, srcw, dstw, zrows)
    h = _gin_mlp(x, p0, W1_0, b1_0, W2_0, b2_0, eps_0)
    p1 = segsum(h, srcw, dstw, zrows)
    emb = _gin_mlp(h, p1, W1_1, b1_1, W2_1, b2_1, eps_1)

    idx = jnp.concatenate([v, a, b])
    g = _make_gather_rows()(emb, idx)
    loss = _head(g, P1, pb1, P2, pb2)
    return loss[0, 0]
